# baseline (speedup 1.0000x reference)
import jax
import jax.numpy as jnp
from jax import lax
from jax.experimental import pallas as pl
from jax.experimental.pallas import tpu as pltpu
from jax.experimental.pallas import tpu_sc as plsc

_L = 16


def _log_sigmoid(x):
    return jnp.minimum(x, 0.0) - jnp.log1p(jnp.exp(-jnp.abs(x)))


def _sc_info():
    try:
        info = plsc.get_sparse_core_info()
        return int(info.num_cores), int(info.num_subcores)
    except Exception:
        return 2, 16


def _bcast(vec, lane):
    idx = jnp.full((_L,), lane, jnp.int32)
    return jnp.take_along_axis(vec, idx, axis=0)


def _sc_scores(tables, idxs, Ltot, E, EE, NU, NP):
    NC, NSC = _sc_info()
    NW = NC * NSC
    rpw = Ltot // NW
    HALF = rpw // 2
    PC = 16
    f32 = jnp.float32
    mesh = plsc.VectorSubcoreMesh(core_axis_name="c", subcore_axis_name="s",
                                  num_cores=NC, num_subcores=NSC)

    out_type = (jax.ShapeDtypeStruct((Ltot, _L), f32),
                jax.ShapeDtypeStruct((Ltot, _L), f32))

    def body(t_ucat, t_ugeo, t_pcat, t_pgeo, t_ctr, t_gtr, t_cpr, t_gpr,
             i_ucat, i_ugeo, i_pcat, i_pgeo, i_cat, i_geo,
             o_cat, o_geo,
             iu_v, ip_v, ic_v, ie_u, ie_p, u_v, p_v, t_v, proj_v, s_v, sem):
        wid = lax.axis_index("s") * NC + lax.axis_index("c")
        base = wid * rpw
        lane_lo = lax.iota(jnp.int32, _L)

        def one_row(rc, proj_ref, crow):
            r = crow + rc
            u_lo = u_v[pl.ds(2 * _L * r, _L)]
            u_hi = u_v[pl.ds(2 * _L * r + _L, _L)]
            p_lo = p_v[pl.ds(2 * _L * r, _L)]
            p_hi = p_v[pl.ds(2 * _L * r + _L, _L)]
            t_lo = t_v[r, pl.ds(0, _L)]
            t_hi = t_v[r, pl.ds(_L, _L)]
            acc_lo = t_lo * p_lo
            acc_hi = t_hi * p_hi
            q_lo = jnp.zeros((_L,), f32)
            q_hi = jnp.zeros((_L,), f32)
            for j in range(2 * _L):
                pj = _bcast(p_lo if j < _L else p_hi, j % _L)
                q_lo = q_lo + pj * proj_ref[rc, pl.ds(2 * _L * j, _L)]
                q_hi = q_hi + pj * proj_ref[rc, pl.ds(2 * _L * j + _L, _L)]
            s_v[r, :] = acc_lo + acc_hi + q_lo * u_lo + q_hi * u_hi

        def build_idx(ie, src, hb, nrows):
            f_lo = lane_lo * nrows
            f_hi = (lane_lo + _L) * nrows

            def brow(rc, cy):
                rv = plsc.load_gather(src, [jnp.full((_L,), hb + rc,
                                                     jnp.int32)])
                ie[pl.ds(2 * _L * rc, _L)] = f_lo + rv
                ie[pl.ds(2 * _L * rc + _L, _L)] = f_hi + rv
                return cy

            lax.fori_loop(0, HALF, brow, 0)

        def side(t_u, t_p, t_t, t_proj, i_u, i_p, i_c, o_s, nu, npoi):
            pltpu.sync_copy(i_u.at[pl.ds(base, rpw)], iu_v)
            pltpu.sync_copy(i_p.at[pl.ds(base, rpw)], ip_v)
            pltpu.sync_copy(i_c.at[pl.ds(base, rpw)], ic_v)
            for h in range(2):
                hb = h * HALF
                build_idx(ie_u, iu_v, hb, nu)
                build_idx(ie_p, ip_v, hb, npoi)
                cu = pltpu.async_copy(t_u.at[ie_u], u_v, sem)
                cp = pltpu.async_copy(t_p.at[ie_p], p_v, sem)
                ct = pltpu.async_copy(t_t.at[ic_v.at[pl.ds(hb, HALF)]],
                                      t_v, sem)
                cu.wait()
                cp.wait()
                ct.wait()

                def chunk(c, carry):
                    crow = c * PC
                    pltpu.async_copy(
                        t_proj.at[ic_v.at[pl.ds(hb + crow, PC)]],
                        proj_v, sem).wait()
                    lax.fori_loop(
                        0, PC,
                        lambda rc, cy: (one_row(rc, proj_v, crow), cy)[1],
                        0)
                    return carry

                lax.fori_loop(0, HALF // PC, chunk, 0)
                pltpu.sync_copy(s_v, o_s.at[pl.ds(base + hb, HALF)])

        side(t_ucat, t_pcat, t_ctr, t_cpr, i_ucat, i_pcat, i_cat, o_cat,
             NU, NP)
        side(t_ugeo, t_pgeo, t_gtr, t_gpr, i_ugeo, i_pgeo, i_geo, o_geo,
             NU, NP)

    call = pl.kernel(
        body,
        out_type=out_type,
        mesh=mesh,
        scratch_types=[
            pltpu.VMEM((rpw,), jnp.int32),
            pltpu.VMEM((rpw,), jnp.int32),
            pltpu.VMEM((rpw,), jnp.int32),
            pltpu.VMEM((HALF * E,), jnp.int32),
            pltpu.VMEM((HALF * E,), jnp.int32),
            pltpu.VMEM((HALF * E,), f32),
            pltpu.VMEM((HALF * E,), f32),
            pltpu.VMEM((HALF, E), f32),
            pltpu.VMEM((PC, EE), f32),
            pltpu.VMEM((HALF, _L), f32),
            pltpu.SemaphoreType.DMA,
        ],
        compiler_params=pltpu.CompilerParams(use_tc_tiling_on_sc=False,
                                             needs_layout_passes=False),
    )
    return call(*tables, *idxs)


def _finale(sc3, sg3, B, NSn):
    f32 = jnp.float32

    def body(sc_ref, sg_ref, pos_ref, neg_ref):
        scv = jnp.sum(sc_ref[...], axis=2)
        sgv = jnp.sum(sg_ref[...], axis=2)
        pos_ref[...] = -(_log_sigmoid(scv[0:1, :]) + _log_sigmoid(sgv[0:1, :]))
        catsum = jnp.sum(_log_sigmoid(-scv[1:, :]), axis=1, keepdims=True)
        geosum = jnp.sum(sgv[1:, :], axis=1, keepdims=True)
        neg_ref[...] = -(catsum + float(B) * _log_sigmoid(-geosum))

    return pl.pallas_call(
        body,
        out_shape=(jax.ShapeDtypeStruct((1, B), f32),
                   jax.ShapeDtypeStruct((NSn, 1), f32)),
    )(sc3, sg3)


def kernel(pos_u, pos_c, pos_p, pos_g, neg_u, neg_c, neg_p, neg_u2, neg_g,
           neg_p2, NS, user_cat_w, user_geo_w, POI_cat_w, POI_geo_w,
           cat_tr_w, cat_proj_w, geo_tr_w, geo_proj_w):
    B = pos_u.shape[0]
    NSn = neg_u.shape[0]
    Ltot = (1 + NSn) * B
    E = user_cat_w.shape[1]
    EE = cat_proj_w.shape[1]

    def flat(p, n):
        return jnp.concatenate([p[None], n], axis=0).reshape(-1).astype(jnp.int32)

    idx_ucat = flat(pos_u, neg_u)
    idx_ugeo = flat(pos_u, neg_u2)
    idx_pcat = flat(pos_p, neg_p)
    idx_pgeo = flat(pos_p, neg_p2)
    idx_cat = flat(pos_c, neg_c)
    idx_geo = flat(pos_g, neg_g)

    s_cat, s_geo = _sc_scores(
        (user_cat_w.T.reshape(-1), user_geo_w.T.reshape(-1),
         POI_cat_w.T.reshape(-1), POI_geo_w.T.reshape(-1),
         cat_tr_w, geo_tr_w, cat_proj_w, geo_proj_w),
        (idx_ucat, idx_ugeo, idx_pcat, idx_pgeo, idx_cat, idx_geo),
        Ltot, E, EE, user_cat_w.shape[0], POI_cat_w.shape[0])

    pos2, neg2 = _finale(s_cat.reshape(1 + NSn, B, _L),
                         s_geo.reshape(1 + NSn, B, _L), B, NSn)
    return pos2.reshape(B), neg2.reshape(NSn)


# speedup vs baseline: 4.8119x; 4.8119x over previous
import jax
import jax.numpy as jnp
from jax import lax
from jax.experimental import pallas as pl
from jax.experimental.pallas import tpu as pltpu
from jax.experimental.pallas import tpu_sc as plsc

_L = 16


def _log_sigmoid(x):
    return jnp.minimum(x, 0.0) - jnp.log1p(jnp.exp(-jnp.abs(x)))


def _sc_info():
    try:
        info = plsc.get_sparse_core_info()
        return int(info.num_cores), int(info.num_subcores)
    except Exception:
        return 2, 16


def _bcast(vec, lane):
    idx = jnp.full((_L,), lane, jnp.int32)
    return jnp.take_along_axis(vec, idx, axis=0)


def _sc_scores(tables, idxs, Ltot, E, EE):
    NC, NSC = _sc_info()
    NW = NC * NSC
    rpw = Ltot // NW
    HALF = rpw // 2
    PC = 16
    f32 = jnp.float32
    mesh = plsc.VectorSubcoreMesh(core_axis_name="c", subcore_axis_name="s",
                                  num_cores=NC, num_subcores=NSC)

    out_type = (jax.ShapeDtypeStruct((Ltot, _L), f32),
                jax.ShapeDtypeStruct((Ltot, _L), f32))

    def body(t_ucat, t_ugeo, t_pcat, t_pgeo, t_ctr, t_gtr, t_cpr, t_gpr,
             i_ucat, i_ugeo, i_pcat, i_pgeo, i_cat, i_geo,
             o_cat, o_geo,
             iu_v, ip_v, ic_v, u_v, p_v, t_v, proj_v, proj2_v, s_v, sem,
             sem2):
        wid = lax.axis_index("s") * NC + lax.axis_index("c")
        base = wid * rpw

        def one_row(rc, proj_ref, crow):
            r = crow + rc
            u_lo = u_v[r, pl.ds(0, _L)]
            u_hi = u_v[r, pl.ds(_L, _L)]
            p_lo = p_v[r, pl.ds(0, _L)]
            p_hi = p_v[r, pl.ds(_L, _L)]
            t_lo = t_v[r, pl.ds(0, _L)]
            t_hi = t_v[r, pl.ds(_L, _L)]
            acc_lo = t_lo * p_lo
            acc_hi = t_hi * p_hi
            q_lo = jnp.zeros((_L,), f32)
            q_hi = jnp.zeros((_L,), f32)
            for j in range(2 * _L):
                pj = _bcast(p_lo if j < _L else p_hi, j % _L)
                q_lo = q_lo + pj * proj_ref[rc, pl.ds(2 * _L * j, _L)]
                q_hi = q_hi + pj * proj_ref[rc, pl.ds(2 * _L * j + _L, _L)]
            s_v[r, :] = acc_lo + acc_hi + q_lo * u_lo + q_hi * u_hi

        def side(t_u, t_p, t_t, t_proj, i_u, i_p, i_c, o_s):
            pltpu.sync_copy(i_u.at[pl.ds(base, rpw)], iu_v)
            pltpu.sync_copy(i_p.at[pl.ds(base, rpw)], ip_v)
            pltpu.sync_copy(i_c.at[pl.ds(base, rpw)], ic_v)
            for h in range(2):
                hb = h * HALF
                cu = pltpu.async_copy(t_u.at[iu_v.at[pl.ds(hb, HALF)]],
                                      u_v, sem)
                cp = pltpu.async_copy(t_p.at[ip_v.at[pl.ds(hb, HALF)]],
                                      p_v, sem)
                ct = pltpu.async_copy(t_t.at[ic_v.at[pl.ds(hb, HALF)]],
                                      t_v, sem)
                cu.wait()
                cp.wait()
                ct.wait()

                def pair(c, carry):
                    c0 = (2 * c) * PC
                    c1 = (2 * c + 1) * PC
                    d0 = pltpu.async_copy(
                        t_proj.at[ic_v.at[pl.ds(hb + c0, PC)]], proj_v, sem)
                    d1 = pltpu.async_copy(
                        t_proj.at[ic_v.at[pl.ds(hb + c1, PC)]], proj2_v,
                        sem2)
                    d0.wait()
                    lax.fori_loop(
                        0, PC,
                        lambda rc, cy: (one_row(rc, proj_v, c0), cy)[1], 0)
                    d1.wait()
                    lax.fori_loop(
                        0, PC,
                        lambda rc, cy: (one_row(rc, proj2_v, c1), cy)[1], 0)
                    return carry

                lax.fori_loop(0, HALF // PC // 2, pair, 0)
                pltpu.sync_copy(s_v, o_s.at[pl.ds(base + hb, HALF)])

        side(t_ucat, t_pcat, t_ctr, t_cpr, i_ucat, i_pcat, i_cat, o_cat)
        side(t_ugeo, t_pgeo, t_gtr, t_gpr, i_ugeo, i_pgeo, i_geo, o_geo)

    call = pl.kernel(
        body,
        out_type=out_type,
        mesh=mesh,
        scratch_types=[
            pltpu.VMEM((rpw,), jnp.int32),
            pltpu.VMEM((rpw,), jnp.int32),
            pltpu.VMEM((rpw,), jnp.int32),
            pltpu.VMEM((HALF, E), f32),
            pltpu.VMEM((HALF, E), f32),
            pltpu.VMEM((HALF, E), f32),
            pltpu.VMEM((PC, EE), f32),
            pltpu.VMEM((PC, EE), f32),
            pltpu.VMEM((HALF, _L), f32),
            pltpu.SemaphoreType.DMA,
            pltpu.SemaphoreType.DMA,
        ],
        compiler_params=pltpu.CompilerParams(use_tc_tiling_on_sc=False),
    )
    return call(*tables, *idxs)


def _finale(sc3, sg3, B, NSn):
    f32 = jnp.float32

    def body(sc_ref, sg_ref, pos_ref, neg_ref):
        scv = jnp.sum(sc_ref[...], axis=2)
        sgv = jnp.sum(sg_ref[...], axis=2)
        pos_ref[...] = -(_log_sigmoid(scv[0:1, :]) + _log_sigmoid(sgv[0:1, :]))
        catsum = jnp.sum(_log_sigmoid(-scv[1:, :]), axis=1, keepdims=True)
        geosum = jnp.sum(sgv[1:, :], axis=1, keepdims=True)
        neg_ref[...] = -(catsum + float(B) * _log_sigmoid(-geosum))

    return pl.pallas_call(
        body,
        out_shape=(jax.ShapeDtypeStruct((1, B), f32),
                   jax.ShapeDtypeStruct((NSn, 1), f32)),
    )(sc3, sg3)


def kernel(pos_u, pos_c, pos_p, pos_g, neg_u, neg_c, neg_p, neg_u2, neg_g,
           neg_p2, NS, user_cat_w, user_geo_w, POI_cat_w, POI_geo_w,
           cat_tr_w, cat_proj_w, geo_tr_w, geo_proj_w):
    B = pos_u.shape[0]
    NSn = neg_u.shape[0]
    Ltot = (1 + NSn) * B
    E = user_cat_w.shape[1]
    EE = cat_proj_w.shape[1]

    def flat(p, n):
        return jnp.concatenate([p[None], n], axis=0).reshape(-1).astype(jnp.int32)

    idx_ucat = flat(pos_u, neg_u)
    idx_ugeo = flat(pos_u, neg_u2)
    idx_pcat = flat(pos_p, neg_p)
    idx_pgeo = flat(pos_p, neg_p2)
    idx_cat = flat(pos_c, neg_c)
    idx_geo = flat(pos_g, neg_g)

    s_cat, s_geo = _sc_scores(
        (user_cat_w, user_geo_w, POI_cat_w, POI_geo_w,
         cat_tr_w, geo_tr_w, cat_proj_w, geo_proj_w),
        (idx_ucat, idx_ugeo, idx_pcat, idx_pgeo, idx_cat, idx_geo),
        Ltot, E, EE)

    pos2, neg2 = _finale(s_cat.reshape(1 + NSn, B, _L),
                         s_geo.reshape(1 + NSn, B, _L), B, NSn)
    return pos2.reshape(B), neg2.reshape(NSn)


# speedup vs baseline: 5.0295x; 1.0452x over previous
import jax
import jax.numpy as jnp
from jax import lax
from jax.experimental import pallas as pl
from jax.experimental.pallas import tpu as pltpu
from jax.experimental.pallas import tpu_sc as plsc

_L = 16


def _log_sigmoid(x):
    return jnp.minimum(x, 0.0) - jnp.log1p(jnp.exp(-jnp.abs(x)))


def _sc_info():
    try:
        info = plsc.get_sparse_core_info()
        return int(info.num_cores), int(info.num_subcores)
    except Exception:
        return 2, 16


def _bcast(vec, lane):
    idx = jnp.full((_L,), lane, jnp.int32)
    return jnp.take_along_axis(vec, idx, axis=0)


def _sc_scores_side(tables, idxs, Ltot, E, EE, name):
    NC, NSC = _sc_info()
    NW = NC * NSC
    rpw = Ltot // NW
    HALF = rpw // 2
    PC = 16
    f32 = jnp.float32
    mesh = plsc.VectorSubcoreMesh(core_axis_name="c", subcore_axis_name="s",
                                  num_cores=NC, num_subcores=NSC)

    out_type = jax.ShapeDtypeStruct((Ltot, _L), f32)

    def body(t_u, t_p, t_t, t_proj,
             i_u, i_p, i_c,
             o_s,
             iu_v, ip_v, ic_v, u_v, p_v, t_v, proj_v, proj2_v, s_v, sem,
             sem2):
        wid = lax.axis_index("s") * NC + lax.axis_index("c")
        base = wid * rpw

        def one_row(rc, proj_ref, crow):
            r = crow + rc
            u_lo = u_v[r, pl.ds(0, _L)]
            u_hi = u_v[r, pl.ds(_L, _L)]
            p_lo = p_v[r, pl.ds(0, _L)]
            p_hi = p_v[r, pl.ds(_L, _L)]
            t_lo = t_v[r, pl.ds(0, _L)]
            t_hi = t_v[r, pl.ds(_L, _L)]
            acc_lo = t_lo * p_lo
            acc_hi = t_hi * p_hi
            q_lo = jnp.zeros((_L,), f32)
            q_hi = jnp.zeros((_L,), f32)
            for j in range(2 * _L):
                pj = _bcast(p_lo if j < _L else p_hi, j % _L)
                q_lo = q_lo + pj * proj_ref[rc, pl.ds(2 * _L * j, _L)]
                q_hi = q_hi + pj * proj_ref[rc, pl.ds(2 * _L * j + _L, _L)]
            s_v[r, :] = acc_lo + acc_hi + q_lo * u_lo + q_hi * u_hi

        if True:
            pltpu.sync_copy(i_u.at[pl.ds(base, rpw)], iu_v)
            pltpu.sync_copy(i_p.at[pl.ds(base, rpw)], ip_v)
            pltpu.sync_copy(i_c.at[pl.ds(base, rpw)], ic_v)
            for h in range(2):
                hb = h * HALF
                cu = pltpu.async_copy(t_u.at[iu_v.at[pl.ds(hb, HALF)]],
                                      u_v, sem)
                cp = pltpu.async_copy(t_p.at[ip_v.at[pl.ds(hb, HALF)]],
                                      p_v, sem)
                ct = pltpu.async_copy(t_t.at[ic_v.at[pl.ds(hb, HALF)]],
                                      t_v, sem)
                cu.wait()
                cp.wait()
                ct.wait()

                def pair(c, carry):
                    c0 = (2 * c) * PC
                    c1 = (2 * c + 1) * PC
                    d0 = pltpu.async_copy(
                        t_proj.at[ic_v.at[pl.ds(hb + c0, PC)]], proj_v, sem)
                    d1 = pltpu.async_copy(
                        t_proj.at[ic_v.at[pl.ds(hb + c1, PC)]], proj2_v,
                        sem2)
                    d0.wait()
                    lax.fori_loop(
                        0, PC,
                        lambda rc, cy: (one_row(rc, proj_v, c0), cy)[1], 0)
                    d1.wait()
                    lax.fori_loop(
                        0, PC,
                        lambda rc, cy: (one_row(rc, proj2_v, c1), cy)[1], 0)
                    return carry

                lax.fori_loop(0, HALF // PC // 2, pair, 0)
                pltpu.sync_copy(s_v, o_s.at[pl.ds(base + hb, HALF)])

    call = pl.kernel(
        body,
        out_type=out_type,
        mesh=mesh,
        scratch_types=[
            pltpu.VMEM((rpw,), jnp.int32),
            pltpu.VMEM((rpw,), jnp.int32),
            pltpu.VMEM((rpw,), jnp.int32),
            pltpu.VMEM((HALF, E), f32),
            pltpu.VMEM((HALF, E), f32),
            pltpu.VMEM((HALF, E), f32),
            pltpu.VMEM((PC, EE), f32),
            pltpu.VMEM((PC, EE), f32),
            pltpu.VMEM((HALF, _L), f32),
            pltpu.SemaphoreType.DMA,
            pltpu.SemaphoreType.DMA,
        ],
        compiler_params=pltpu.CompilerParams(use_tc_tiling_on_sc=False),
        name=name,
    )
    return call(*tables, *idxs)


def _finale(sc3, sg3, B, NSn):
    f32 = jnp.float32

    def body(sc_ref, sg_ref, pos_ref, neg_ref):
        scv = jnp.sum(sc_ref[...], axis=2)
        sgv = jnp.sum(sg_ref[...], axis=2)
        pos_ref[...] = -(_log_sigmoid(scv[0:1, :]) + _log_sigmoid(sgv[0:1, :]))
        catsum = jnp.sum(_log_sigmoid(-scv[1:, :]), axis=1, keepdims=True)
        geosum = jnp.sum(sgv[1:, :], axis=1, keepdims=True)
        neg_ref[...] = -(catsum + float(B) * _log_sigmoid(-geosum))

    return pl.pallas_call(
        body,
        out_shape=(jax.ShapeDtypeStruct((1, B), f32),
                   jax.ShapeDtypeStruct((NSn, 1), f32)),
    )(sc3, sg3)


def kernel(pos_u, pos_c, pos_p, pos_g, neg_u, neg_c, neg_p, neg_u2, neg_g,
           neg_p2, NS, user_cat_w, user_geo_w, POI_cat_w, POI_geo_w,
           cat_tr_w, cat_proj_w, geo_tr_w, geo_proj_w):
    B = pos_u.shape[0]
    NSn = neg_u.shape[0]
    Ltot = (1 + NSn) * B
    E = user_cat_w.shape[1]
    EE = cat_proj_w.shape[1]

    def flat(p, n):
        return jnp.concatenate([p[None], n], axis=0).reshape(-1).astype(jnp.int32)

    idx_ucat = flat(pos_u, neg_u)
    idx_ugeo = flat(pos_u, neg_u2)
    idx_pcat = flat(pos_p, neg_p)
    idx_pgeo = flat(pos_p, neg_p2)
    idx_cat = flat(pos_c, neg_c)
    idx_geo = flat(pos_g, neg_g)

    s_cat = _sc_scores_side(
        (user_cat_w, POI_cat_w, cat_tr_w, cat_proj_w),
        (idx_ucat, idx_pcat, idx_cat), Ltot, E, EE, "sc_cat_scores")
    s_geo = _sc_scores_side(
        (user_geo_w, POI_geo_w, geo_tr_w, geo_proj_w),
        (idx_ugeo, idx_pgeo, idx_geo), Ltot, E, EE, "sc_geo_scores")

    pos2, neg2 = _finale(s_cat.reshape(1 + NSn, B, _L),
                         s_geo.reshape(1 + NSn, B, _L), B, NSn)
    return pos2.reshape(B), neg2.reshape(NSn)
